# baseline (speedup 1.0000x reference)
import jax
import jax.numpy as jnp
from jax.experimental import pallas as pl
from jax.experimental.pallas import tpu as pltpu

N = 4096
F32 = jnp.float32
BF16 = jnp.bfloat16
BLK = 256
NB = N // BLK
BLKK = 256
NBK = N // BLKK
BLK_E = 512
NB_E = N // BLK_E
BLK_S = 1024


def _dot(a, b):
    return jnp.dot(a.astype(BF16), b.astype(BF16),
                   preferred_element_type=F32)


def _xw_kernel(x_ref, w1_ref, wg1a_ref, wg1b_ref, os_ref, of_ref):
    x = x_ref[...]
    xw1 = _dot(x, w1_ref[...]).astype(BF16)
    os_ref[:, :256] = xw1
    os_ref[:, 256:512] = _dot(x, wg1a_ref[...]).astype(BF16)
    of_ref[:, :256] = xw1
    of_ref[:, 256:512] = _dot(x, wg1b_ref[...]).astype(BF16)


def _kf_kernel(f_ref, xwf_ref, b1_ref, bg1b_ref, wg2b_ref, bg_ref,
               fh1_ref, h2_ref, h2t_ref, e2_ref,
               fadj_bf, r2s):
    g = pl.program_id(0)
    i = pl.program_id(1)
    rows = pl.ds(i * BLK, BLK)

    @pl.when(g == 0)
    def _phase0():
        fadj_bf[rows, :] = f_ref[...].astype(BF16)
        fb = fadj_bf[rows, :]
        xwf = xwf_ref[...]
        fh1 = jax.nn.relu(_dot(fb, xwf[:, :256]) + b1_ref[...])
        t2 = jax.nn.relu(_dot(fb, xwf[:, 256:512]) + bg1b_ref[...])
        fh1_ref[...] = fh1.astype(BF16)
        r2s[rows, :] = _dot(t2.astype(BF16), wg2b_ref[...]).astype(BF16)

    @pl.when(g == 1)
    def _phase1():
        fb = fadj_bf[rows, :]
        h2b = jax.nn.relu(_dot(fb, r2s[...]) + bg_ref[...])
        h2_ref[...] = h2b.astype(BF16)
        h2t_ref[...] = h2b.T.astype(BF16)
        n2 = jnp.sqrt(jnp.sum(h2b * h2b, axis=1, keepdims=True))
        e2_ref[...] = h2b / n2


def _ks_kernel(s_ref, xws_ref, b1_ref, bg1a_ref, fh1_ref, w2_ref, w3_ref,
               wg2a_ref, b2_ref, b3_ref, bg2a_ref,
               smu_ref, slv_ref, fmu_ref, flv_ref, h1_ref, h1t_ref,
               sadj_bf, r1s):
    g = pl.program_id(0)
    i = pl.program_id(1)
    rows = pl.ds(i * BLKK, BLKK)

    @pl.when(g == 0)
    def _phase0():
        sadj_bf[rows, :] = s_ref[...].astype(BF16)
        sb = sadj_bf[rows, :]
        xws = xws_ref[...]
        pa = _dot(sb, xws)
        sh1 = jax.nn.relu(pa[:, :256] + b1_ref[...]).astype(BF16)
        t1 = jax.nn.relu(pa[:, 256:512] + bg1a_ref[...]).astype(BF16)
        fh1 = fh1_ref[...]
        r1s[rows, 0:128] = _dot(sh1, w2_ref[...]).astype(BF16)
        r1s[rows, 128:256] = _dot(sh1, w3_ref[...]).astype(BF16)
        r1s[rows, 256:384] = _dot(fh1, w2_ref[...]).astype(BF16)
        r1s[rows, 384:512] = _dot(fh1, w3_ref[...]).astype(BF16)
        r1s[rows, 512:640] = _dot(t1, wg2a_ref[...]).astype(BF16)

    @pl.when(g == 1)
    def _phase1():
        sb = sadj_bf[rows, :]
        p = _dot(sb, r1s[...])
        smu_ref[...] = jax.nn.relu(p[:, 0:128] + b2_ref[...])
        slv_ref[...] = jax.nn.relu(p[:, 128:256] + b3_ref[...])
        fmu_ref[...] = jax.nn.relu(p[:, 256:384] + b2_ref[...])
        flv_ref[...] = jax.nn.relu(p[:, 384:512] + b3_ref[...])
        h1b = jax.nn.relu(p[:, 512:640] + bg2a_ref[...])
        h1_ref[...] = h1b
        h1t_ref[...] = h1b.T.astype(BF16)


def _ke_kernel(h1_ref, h2_ref, h1t_ref, h2t_ref, m1_ref, m2_ref, bm2_ref,
               m3_ref, bm3_ref, ls_ref,
               a1_ref, a2_ref, e1_ref, out_ref, els_ref):
    r1 = h1_ref[...]
    a1_ref[...] = jax.nn.sigmoid(_dot(r1, h1t_ref[...]))
    a2_ref[...] = jax.nn.sigmoid(_dot(h2_ref[...], h2t_ref[...]))
    n1 = jnp.sqrt(jnp.sum(r1 * r1, axis=1, keepdims=True))
    e1_ref[...] = r1 / n1
    z = jnp.concatenate([r1.astype(BF16), h2_ref[...]], axis=1)
    t = _dot(z, m1_ref[...])
    t = _dot(t, m2_ref[...]) + bm2_ref[...]
    t = _dot(t, m3_ref[...]) + bm3_ref[...]
    m = jnp.max(t, axis=1, keepdims=True)
    out_ref[...] = t - m - jnp.log(jnp.sum(jnp.exp(t - m), axis=1,
                                           keepdims=True))
    els_ref[...] = jnp.exp(ls_ref[...])


def kernel(x, sadj, fadj, W1, b1, W2, b2, W3, b3, Wg1a, bg1a, Wg2a, bg2a,
           Wg1b, bg1b, Wg2b, bg2b, M1, M2, bM2, M3, bM3, logit_scale):
    XWs, XWf = pl.pallas_call(
        _xw_kernel,
        grid=(N // BLK_S,),
        in_specs=[
            pl.BlockSpec((BLK_S, 512), lambda i: (i, 0)),
            pl.BlockSpec((512, 256), lambda i: (0, 0)),
            pl.BlockSpec((512, 256), lambda i: (0, 0)),
            pl.BlockSpec((512, 256), lambda i: (0, 0)),
        ],
        out_specs=[pl.BlockSpec((BLK_S, 512), lambda i: (i, 0)),
                   pl.BlockSpec((BLK_S, 512), lambda i: (i, 0))],
        out_shape=[jax.ShapeDtypeStruct((N, 512), BF16),
                   jax.ShapeDtypeStruct((N, 512), BF16)],
        compiler_params=pltpu.CompilerParams(
            dimension_semantics=("parallel",)),
    )(x, W1, Wg1a, Wg1b)

    last = NB - 1
    adj_spec = pl.BlockSpec((BLK, N),
                            lambda g, i: (jnp.where(g == 0, i, last), 0))
    res2 = lambda shape: pl.BlockSpec(shape, lambda g, i: (0, 0))
    p0b = lambda w: pl.BlockSpec((BLK, w),
                                 lambda g, i: (jnp.where(g == 0, i, last), 0))
    p1b = lambda w: pl.BlockSpec((BLK, w),
                                 lambda g, i: (jnp.where(g == 1, i, 0), 0))
    p1t = pl.BlockSpec((128, BLK),
                       lambda g, i: (0, jnp.where(g == 1, i, 0)))
    arb2 = pltpu.CompilerParams(
        dimension_semantics=("arbitrary", "arbitrary"),
        vmem_limit_bytes=100 * 1024 * 1024)

    b1r = b1.reshape(1, 256)
    fh1, h2, h2t, emb2 = pl.pallas_call(
        _kf_kernel,
        grid=(2, NB),
        in_specs=[adj_spec, res2((N, 512)), res2((1, 256)), res2((1, 256)),
                  res2((256, 128)), res2((1, 128))],
        out_specs=[p0b(256), p1b(128), p1t, p1b(128)],
        out_shape=[jax.ShapeDtypeStruct((N, 256), BF16),
                   jax.ShapeDtypeStruct((N, 128), BF16),
                   jax.ShapeDtypeStruct((128, N), BF16),
                   jax.ShapeDtypeStruct((N, 128), F32)],
        scratch_shapes=[pltpu.VMEM((N, N), BF16),
                        pltpu.VMEM((N, 128), BF16)],
        compiler_params=arb2,
    )(fadj, XWf, b1r, bg1b.reshape(1, 256), Wg2b, bg2b.reshape(1, 128))

    lastk = NBK - 1
    adjk_spec = pl.BlockSpec((BLKK, N),
                             lambda g, i: (jnp.where(g == 0, i, lastk), 0))
    pk0b = lambda w: pl.BlockSpec((BLKK, w),
                                  lambda g, i: (jnp.where(g == 0, i, lastk),
                                                0))
    pk1b = lambda w: pl.BlockSpec((BLKK, w),
                                  lambda g, i: (jnp.where(g == 1, i, 0), 0))
    pk1t = pl.BlockSpec((128, BLKK),
                        lambda g, i: (0, jnp.where(g == 1, i, 0)))
    smu, slv, fmu, flv, h1, h1t = pl.pallas_call(
        _ks_kernel,
        grid=(2, NBK),
        in_specs=[adjk_spec, res2((N, 512)), res2((1, 256)), res2((1, 256)),
                  pk0b(256), res2((256, 128)), res2((256, 128)),
                  res2((256, 128)), res2((1, 128)), res2((1, 128)),
                  res2((1, 128))],
        out_specs=[pk1b(128), pk1b(128), pk1b(128), pk1b(128), pk1b(128),
                   pk1t],
        out_shape=[jax.ShapeDtypeStruct((N, 128), F32)] * 5 +
                  [jax.ShapeDtypeStruct((128, N), BF16)],
        scratch_shapes=[pltpu.VMEM((N, N), BF16),
                        pltpu.VMEM((N, 640), BF16)],
        compiler_params=arb2,
    )(sadj, XWs, b1r, bg1a.reshape(1, 256), fh1, W2, W3, Wg2a,
      b2.reshape(1, 128), b3.reshape(1, 128), bg2a.reshape(1, 128))

    he = pl.BlockSpec((BLK_E, 128), lambda i: (i, 0))
    res = lambda shape: pl.BlockSpec(shape, lambda i: (0, 0))
    A1, A2, emb1, out, els = pl.pallas_call(
        _ke_kernel,
        grid=(NB_E,),
        in_specs=[he, he, res((128, N)), res((128, N)), res((256, 256)),
                  res((256, 128)), res((1, 128)), res((128, 16)),
                  res((1, 16)), res((1, 1))],
        out_specs=[pl.BlockSpec((BLK_E, N), lambda i: (i, 0)),
                   pl.BlockSpec((BLK_E, N), lambda i: (i, 0)),
                   he,
                   pl.BlockSpec((BLK_E, 16), lambda i: (i, 0)),
                   pl.BlockSpec((1, 1), lambda i: (0, 0))],
        out_shape=[jax.ShapeDtypeStruct((N, N), F32),
                   jax.ShapeDtypeStruct((N, N), F32),
                   jax.ShapeDtypeStruct((N, 128), F32),
                   jax.ShapeDtypeStruct((N, 16), F32),
                   jax.ShapeDtypeStruct((1, 1), F32)],
        compiler_params=pltpu.CompilerParams(
            dimension_semantics=("parallel",),
            vmem_limit_bytes=100 * 1024 * 1024),
    )(h1, h2, h1t, h2t, M1, M2, bM2.reshape(1, 128), M3,
      bM3.reshape(1, 16), logit_scale.reshape(1, 1))

    return (out, A1, A2, emb1, emb2, els.reshape(()), smu, slv, fmu, flv)


# speedup vs baseline: 1.0506x; 1.0506x over previous
import jax
import jax.numpy as jnp
from jax.experimental import pallas as pl
from jax.experimental.pallas import tpu as pltpu

N = 4096
F32 = jnp.float32
BF16 = jnp.bfloat16
BLK = 512
NB = N // BLK
BLKK = 256
NBK = N // BLKK
BLK_E = 512
NB_E = N // BLK_E
BLK_S = 1024


def _dot(a, b):
    return jnp.dot(a.astype(BF16), b.astype(BF16),
                   preferred_element_type=F32)


def _xw_kernel(x_ref, w1_ref, wg1a_ref, wg1b_ref, o1_ref, oa_ref, ob_ref):
    x = x_ref[...]
    o1_ref[...] = _dot(x, w1_ref[...]).astype(BF16)
    oa_ref[...] = _dot(x, wg1a_ref[...]).astype(BF16)
    ob_ref[...] = _dot(x, wg1b_ref[...]).astype(BF16)


def _kf_kernel(f_ref, xw1_ref, xwb_ref, b1_ref, bg1b_ref, wg2b_ref, bg_ref,
               fh1_ref, h2_ref, h2t_ref, e2_ref,
               fadj_bf, r2s):
    g = pl.program_id(0)
    i = pl.program_id(1)
    rows = pl.ds(i * BLK, BLK)

    @pl.when(g == 0)
    def _phase0():
        fadj_bf[rows, :] = f_ref[...].astype(BF16)
        fb = fadj_bf[rows, :]
        fh1 = jax.nn.relu(_dot(fb, xw1_ref[...]) + b1_ref[...])
        t2 = jax.nn.relu(_dot(fb, xwb_ref[...]) + bg1b_ref[...])
        fh1_ref[...] = fh1.astype(BF16)
        r2s[rows, :] = _dot(t2.astype(BF16), wg2b_ref[...]).astype(BF16)

    @pl.when(g == 1)
    def _phase1():
        fb = fadj_bf[rows, :]
        h2b = jax.nn.relu(_dot(fb, r2s[...]) + bg_ref[...])
        h2_ref[...] = h2b.astype(BF16)
        h2t_ref[...] = h2b.T.astype(BF16)
        n2 = jnp.sqrt(jnp.sum(h2b * h2b, axis=1, keepdims=True))
        e2_ref[...] = h2b / n2


def _ks_kernel(s_ref, xw1_ref, xwa_ref, b1_ref, bg1a_ref, fh1_ref, w2_ref, w3_ref,
               wg2a_ref, b2_ref, b3_ref, bg2a_ref,
               smu_ref, slv_ref, fmu_ref, flv_ref, h1_ref, h1t_ref,
               sadj_bf, r1s):
    g = pl.program_id(0)
    i = pl.program_id(1)
    rows = pl.ds(i * BLKK, BLKK)

    @pl.when(g == 0)
    def _phase0():
        sadj_bf[rows, :] = s_ref[...].astype(BF16)
        sb = sadj_bf[rows, :]
        sh1 = jax.nn.relu(_dot(sb, xw1_ref[...]) + b1_ref[...]).astype(BF16)
        t1 = jax.nn.relu(_dot(sb, xwa_ref[...]) + bg1a_ref[...]).astype(BF16)
        fh1 = fh1_ref[...]
        r1s[rows, 0:128] = _dot(sh1, w2_ref[...]).astype(BF16)
        r1s[rows, 128:256] = _dot(sh1, w3_ref[...]).astype(BF16)
        r1s[rows, 256:384] = _dot(fh1, w2_ref[...]).astype(BF16)
        r1s[rows, 384:512] = _dot(fh1, w3_ref[...]).astype(BF16)
        r1s[rows, 512:640] = _dot(t1, wg2a_ref[...]).astype(BF16)

    @pl.when(g == 1)
    def _phase1():
        sb = sadj_bf[rows, :]
        p = _dot(sb, r1s[...])
        smu_ref[...] = jax.nn.relu(p[:, 0:128] + b2_ref[...])
        slv_ref[...] = jax.nn.relu(p[:, 128:256] + b3_ref[...])
        fmu_ref[...] = jax.nn.relu(p[:, 256:384] + b2_ref[...])
        flv_ref[...] = jax.nn.relu(p[:, 384:512] + b3_ref[...])
        h1b = jax.nn.relu(p[:, 512:640] + bg2a_ref[...])
        h1_ref[...] = h1b
        h1t_ref[...] = h1b.T.astype(BF16)


def _ke_kernel(h1_ref, h2_ref, h1t_ref, h2t_ref, m1_ref, m2_ref, bm2_ref,
               m3_ref, bm3_ref, ls_ref,
               a1_ref, a2_ref, e1_ref, out_ref, els_ref):
    r1 = h1_ref[...]
    a1_ref[...] = jax.nn.sigmoid(_dot(r1, h1t_ref[...]))
    a2_ref[...] = jax.nn.sigmoid(_dot(h2_ref[...], h2t_ref[...]))
    n1 = jnp.sqrt(jnp.sum(r1 * r1, axis=1, keepdims=True))
    e1_ref[...] = r1 / n1
    z = jnp.concatenate([r1.astype(BF16), h2_ref[...]], axis=1)
    t = _dot(z, m1_ref[...])
    t = _dot(t, m2_ref[...]) + bm2_ref[...]
    t = _dot(t, m3_ref[...]) + bm3_ref[...]
    m = jnp.max(t, axis=1, keepdims=True)
    out_ref[...] = t - m - jnp.log(jnp.sum(jnp.exp(t - m), axis=1,
                                           keepdims=True))
    els_ref[...] = jnp.exp(ls_ref[...])


def kernel(x, sadj, fadj, W1, b1, W2, b2, W3, b3, Wg1a, bg1a, Wg2a, bg2a,
           Wg1b, bg1b, Wg2b, bg2b, M1, M2, bM2, M3, bM3, logit_scale):
    XW1, XWa, XWb = pl.pallas_call(
        _xw_kernel,
        grid=(N // BLK_S,),
        in_specs=[
            pl.BlockSpec((BLK_S, 512), lambda i: (i, 0)),
            pl.BlockSpec((512, 256), lambda i: (0, 0)),
            pl.BlockSpec((512, 256), lambda i: (0, 0)),
            pl.BlockSpec((512, 256), lambda i: (0, 0)),
        ],
        out_specs=[pl.BlockSpec((BLK_S, 256), lambda i: (i, 0)),
                   pl.BlockSpec((BLK_S, 256), lambda i: (i, 0)),
                   pl.BlockSpec((BLK_S, 256), lambda i: (i, 0))],
        out_shape=[jax.ShapeDtypeStruct((N, 256), BF16),
                   jax.ShapeDtypeStruct((N, 256), BF16),
                   jax.ShapeDtypeStruct((N, 256), BF16)],
        compiler_params=pltpu.CompilerParams(
            dimension_semantics=("parallel",)),
    )(x, W1, Wg1a, Wg1b)

    last = NB - 1
    adj_spec = pl.BlockSpec((BLK, N),
                            lambda g, i: (jnp.where(g == 0, i, last), 0))
    res2 = lambda shape: pl.BlockSpec(shape, lambda g, i: (0, 0))
    p0b = lambda w: pl.BlockSpec((BLK, w),
                                 lambda g, i: (jnp.where(g == 0, i, last), 0))
    p1b = lambda w: pl.BlockSpec((BLK, w),
                                 lambda g, i: (jnp.where(g == 1, i, 0), 0))
    p1t = pl.BlockSpec((128, BLK),
                       lambda g, i: (0, jnp.where(g == 1, i, 0)))
    arb2 = pltpu.CompilerParams(
        dimension_semantics=("arbitrary", "arbitrary"),
        vmem_limit_bytes=100 * 1024 * 1024)

    b1r = b1.reshape(1, 256)
    fh1, h2, h2t, emb2 = pl.pallas_call(
        _kf_kernel,
        grid=(2, NB),
        in_specs=[adj_spec, res2((N, 256)), res2((N, 256)), res2((1, 256)),
                  res2((1, 256)), res2((256, 128)), res2((1, 128))],
        out_specs=[p0b(256), p1b(128), p1t, p1b(128)],
        out_shape=[jax.ShapeDtypeStruct((N, 256), BF16),
                   jax.ShapeDtypeStruct((N, 128), BF16),
                   jax.ShapeDtypeStruct((128, N), BF16),
                   jax.ShapeDtypeStruct((N, 128), F32)],
        scratch_shapes=[pltpu.VMEM((N, N), BF16),
                        pltpu.VMEM((N, 128), BF16)],
        compiler_params=arb2,
    )(fadj, XW1, XWb, b1r, bg1b.reshape(1, 256), Wg2b, bg2b.reshape(1, 128))

    lastk = NBK - 1
    adjk_spec = pl.BlockSpec((BLKK, N),
                             lambda g, i: (jnp.where(g == 0, i, lastk), 0))
    pk0b = lambda w: pl.BlockSpec((BLKK, w),
                                  lambda g, i: (jnp.where(g == 0, i, lastk),
                                                0))
    pk1b = lambda w: pl.BlockSpec((BLKK, w),
                                  lambda g, i: (jnp.where(g == 1, i, 0), 0))
    pk1t = pl.BlockSpec((128, BLKK),
                        lambda g, i: (0, jnp.where(g == 1, i, 0)))
    smu, slv, fmu, flv, h1, h1t = pl.pallas_call(
        _ks_kernel,
        grid=(2, NBK),
        in_specs=[adjk_spec, res2((N, 256)), res2((N, 256)), res2((1, 256)),
                  res2((1, 256)), pk0b(256), res2((256, 128)),
                  res2((256, 128)), res2((256, 128)), res2((1, 128)),
                  res2((1, 128)), res2((1, 128))],
        out_specs=[pk1b(128), pk1b(128), pk1b(128), pk1b(128), pk1b(128),
                   pk1t],
        out_shape=[jax.ShapeDtypeStruct((N, 128), F32)] * 5 +
                  [jax.ShapeDtypeStruct((128, N), BF16)],
        scratch_shapes=[pltpu.VMEM((N, N), BF16),
                        pltpu.VMEM((N, 640), BF16)],
        compiler_params=arb2,
    )(sadj, XW1, XWa, b1r, bg1a.reshape(1, 256), fh1, W2, W3, Wg2a,
      b2.reshape(1, 128), b3.reshape(1, 128), bg2a.reshape(1, 128))

    he = pl.BlockSpec((BLK_E, 128), lambda i: (i, 0))
    res = lambda shape: pl.BlockSpec(shape, lambda i: (0, 0))
    A1, A2, emb1, out, els = pl.pallas_call(
        _ke_kernel,
        grid=(NB_E,),
        in_specs=[he, he, res((128, N)), res((128, N)), res((256, 256)),
                  res((256, 128)), res((1, 128)), res((128, 16)),
                  res((1, 16)), res((1, 1))],
        out_specs=[pl.BlockSpec((BLK_E, N), lambda i: (i, 0)),
                   pl.BlockSpec((BLK_E, N), lambda i: (i, 0)),
                   he,
                   pl.BlockSpec((BLK_E, 16), lambda i: (i, 0)),
                   pl.BlockSpec((1, 1), lambda i: (0, 0))],
        out_shape=[jax.ShapeDtypeStruct((N, N), F32),
                   jax.ShapeDtypeStruct((N, N), F32),
                   jax.ShapeDtypeStruct((N, 128), F32),
                   jax.ShapeDtypeStruct((N, 16), F32),
                   jax.ShapeDtypeStruct((1, 1), F32)],
        compiler_params=pltpu.CompilerParams(
            dimension_semantics=("parallel",),
            vmem_limit_bytes=100 * 1024 * 1024),
    )(h1, h2, h1t, h2t, M1, M2, bM2.reshape(1, 128), M3,
      bM3.reshape(1, 16), logit_scale.reshape(1, 1))

    return (out, A1, A2, emb1, emb2, els.reshape(()), smu, slv, fmu, flv)
